# Initial kernel scaffold; baseline (speedup 1.0000x reference)
import jax
import jax.numpy as jnp
from jax.experimental import pallas as pl


def kernel(x, edge_index, edge_attr, Wx, bx, We, be, W1, b1, a1, W2, b2, a2, W3, b3):
    raise NotImplementedError("write your pallas kernel here")


# speedup vs baseline: 1.9739x; 1.9739x over previous
import functools

import jax
import jax.numpy as jnp
from jax import lax
from jax.experimental import pallas as pl
from jax.experimental.pallas import tpu as pltpu
from jax.experimental.pallas import tpu_sc as plsc

_NC = 2
_NS = 16
_CHUNK = 128


def _row_block(n, candidates):
    for r in candidates:
        if n % r == 0:
            return r
    return n


def _lin_full_body(x_ref, w_ref, b_ref, full_ref, halves_ref):
    t = jnp.dot(x_ref[...], w_ref[...], preferred_element_type=jnp.float32)
    t = t + b_ref[...]
    full_ref[...] = t
    h = t.shape[1] // 2
    halves_ref[0] = t[:, :h]
    halves_ref[1] = t[:, h:]


def _lin_half_body(x_ref, w_ref, b_ref, halves_ref):
    t = jnp.dot(x_ref[...], w_ref[...], preferred_element_type=jnp.float32)
    t = t + b_ref[...]
    h = t.shape[1] // 2
    halves_ref[0] = t[:, :h]
    halves_ref[1] = t[:, h:]


def _mlp_body(h_ref, agg_ref, w1_ref, b1_ref, a1_ref, w2_ref, b2_ref, a2_ref,
              w3_ref, b3_ref, z_ref):
    a = h_ref[...] + jnp.concatenate([agg_ref[0], agg_ref[1]], axis=1)
    t = jnp.dot(a, w1_ref[...], preferred_element_type=jnp.float32) + b1_ref[...]
    t = jnp.where(t >= 0, t, a1_ref[0, 0] * t)
    t = jnp.dot(t, w2_ref[...], preferred_element_type=jnp.float32) + b2_ref[...]
    t = jnp.where(t >= 0, t, a2_ref[0, 0] * t)
    t = jnp.dot(t, w3_ref[...], preferred_element_type=jnp.float32) + b3_ref[...]
    z_ref[...] = t


def _build_sc_edge1(N, E, H2):
    PT = -(-E // (_NS * _CHUNK)) * _CHUNK
    nh = H2 // 16
    mesh = plsc.VectorSubcoreMesh(core_axis_name="c", subcore_axis_name="s",
                                  num_cores=_NC, num_subcores=_NS)

    @functools.partial(
        pl.kernel,
        out_type=(jax.ShapeDtypeStruct((_NC * E, H2), jnp.float32),
                  jax.ShapeDtypeStruct((_NC * N, H2), jnp.float32)),
        mesh=mesh,
        scratch_types=[
            pltpu.VMEM((1, _CHUNK), jnp.int32),
            pltpu.VMEM((1, _CHUNK), jnp.int32),
            pltpu.VMEM((1, _CHUNK), jnp.int32),
            pltpu.VMEM((1, _CHUNK), jnp.int32),
            pltpu.VMEM((_CHUNK, H2), jnp.float32),
            pltpu.VMEM((_CHUNK, H2), jnp.float32),
            pltpu.VMEM((_CHUNK, H2), jnp.float32),
            pltpu.VMEM((_CHUNK, H2), jnp.float32),
            pltpu.VMEM_SHARED((N, H2), jnp.float32),
            pltpu.SemaphoreType.DMA,
        ],
        compiler_params=pltpu.CompilerParams(use_tc_tiling_on_sc=False),
    )
    def edge1(hh, el, srcr, dstr, es_out, agg_out,
              sidx, didx, soff, doff, hs, hd, elb, zbuf, accum, sem):
        c = lax.axis_index("c")
        s = lax.axis_index("s")
        coff = c * N

        rb0 = (s * N // _NS) // 8 * 8
        rb1 = ((s + 1) * N // _NS) // 8 * 8
        rows = rb1 - rb0
        nfull = rows // _CHUNK
        ntail = (rows % _CHUNK) // 8

        def zrow(r, _):
            for k in range(nh):
                zbuf[r, pl.ds(k * 16, 16)] = jnp.zeros((16,), jnp.float32)
            return 0
        lax.fori_loop(0, _CHUNK, zrow, 0)

        def zfull(t, _):
            pltpu.sync_copy(zbuf, accum.at[pl.ds(rb0 + t * _CHUNK, _CHUNK)])
            return 0
        lax.fori_loop(0, nfull, zfull, 0)

        def ztail(t, _):
            pltpu.sync_copy(zbuf.at[pl.ds(0, 8)],
                            accum.at[pl.ds(rb0 + nfull * _CHUNK + t * 8, 8)])
            return 0
        lax.fori_loop(0, ntail, ztail, 0)
        plsc.subcore_barrier()

        base = s * PT
        nch = jnp.minimum(PT, E - base) // _CHUNK

        def chunk(j, _):
            eb = base + j * _CHUNK
            pltpu.sync_copy(srcr.at[pl.ds(eb, _CHUNK)], sidx.at[0])
            pltpu.sync_copy(dstr.at[pl.ds(eb, _CHUNK)], didx.at[0])
            for k in range(_CHUNK // 16):
                sl = pl.ds(k * 16, 16)
                soff[0, sl] = sidx[0, sl] + coff
                doff[0, sl] = didx[0, sl] + coff
            cp1 = pltpu.make_async_copy(hh.at[soff.at[0]], hs, sem)
            cp2 = pltpu.make_async_copy(hh.at[doff.at[0]], hd, sem)
            cp3 = pltpu.make_async_copy(el.at[pl.ds(c * E + eb, _CHUNK)], elb, sem)
            cp1.start()
            cp2.start()
            cp3.start()
            cp1.wait()
            cp2.wait()
            cp3.wait()

            def row(r, _):
                for k in range(nh):
                    sl = pl.ds(k * 16, 16)
                    av = hs[r, sl]
                    ev = (av + hd[r, sl] + elb[r, sl]) * jnp.float32(1.0 / 3.0)
                    elb[r, sl] = ev
                    hd[r, sl] = jnp.maximum(av + ev, 0.0)
                return 0
            lax.fori_loop(0, _CHUNK, row, 0)
            pltpu.sync_copy(elb, es_out.at[pl.ds(c * E + eb, _CHUNK)])
            pltpu.sync_copy(hd, accum.at[didx.at[0]], add=True)
            return 0
        lax.fori_loop(0, nch, chunk, 0)
        plsc.subcore_barrier()

        def cfull(t, _):
            sl = pl.ds(rb0 + t * _CHUNK, _CHUNK)
            pltpu.sync_copy(accum.at[sl],
                            agg_out.at[pl.ds(coff + rb0 + t * _CHUNK, _CHUNK)])
            return 0
        lax.fori_loop(0, nfull, cfull, 0)

        def ctail(t, _):
            o = rb0 + nfull * _CHUNK + t * 8
            pltpu.sync_copy(accum.at[pl.ds(o, 8)], agg_out.at[pl.ds(coff + o, 8)])
            return 0
        lax.fori_loop(0, ntail, ctail, 0)

    return edge1


def _build_sc_edge2(N, E, HID):
    NW = _NC * _NS
    PT = -(-E // (NW * _CHUNK)) * _CHUNK
    nh = HID // 16
    H2 = HID // 2
    mesh = plsc.VectorSubcoreMesh(core_axis_name="c", subcore_axis_name="s",
                                  num_cores=_NC, num_subcores=_NS)

    @functools.partial(
        pl.kernel,
        out_type=jax.ShapeDtypeStruct((E, HID), jnp.float32),
        mesh=mesh,
        scratch_types=[
            pltpu.VMEM((1, _CHUNK), jnp.int32),
            pltpu.VMEM((1, _CHUNK), jnp.int32),
            pltpu.VMEM((_CHUNK, HID), jnp.float32),
            pltpu.VMEM((_CHUNK, HID), jnp.float32),
            pltpu.VMEM((_CHUNK, H2), jnp.float32),
            pltpu.VMEM((_CHUNK, H2), jnp.float32),
            pltpu.SemaphoreType.DMA,
        ],
        compiler_params=pltpu.CompilerParams(use_tc_tiling_on_sc=False),
    )
    def edge2(zz, es, srcr, dstr, e2_out, sidx, didx, zs, zd, esa, esb, sem):
        c = lax.axis_index("c")
        s = lax.axis_index("s")
        w = s * _NC + c
        base = w * PT
        nch = jnp.maximum(jnp.minimum(PT, E - base), 0) // _CHUNK

        def chunk(j, _):
            eb = base + j * _CHUNK
            pltpu.sync_copy(srcr.at[pl.ds(eb, _CHUNK)], sidx.at[0])
            pltpu.sync_copy(dstr.at[pl.ds(eb, _CHUNK)], didx.at[0])
            cp1 = pltpu.make_async_copy(zz.at[sidx.at[0]], zs, sem)
            cp2 = pltpu.make_async_copy(zz.at[didx.at[0]], zd, sem)
            cp3 = pltpu.make_async_copy(es.at[pl.ds(eb, _CHUNK)], esa, sem)
            cp4 = pltpu.make_async_copy(es.at[pl.ds(E + eb, _CHUNK)], esb, sem)
            cp1.start()
            cp2.start()
            cp3.start()
            cp4.start()
            cp1.wait()
            cp2.wait()
            cp3.wait()
            cp4.wait()

            def row(r, _):
                for k in range(nh):
                    sl = pl.ds(k * 16, 16)
                    if k < nh // 2:
                        ev = esa[r, pl.ds(k * 16, 16)]
                    else:
                        ev = esb[r, pl.ds((k - nh // 2) * 16, 16)]
                    zs[r, sl] = (zs[r, sl] + zd[r, sl] + ev) * jnp.float32(1.0 / 3.0)
                return 0
            lax.fori_loop(0, _CHUNK, row, 0)
            pltpu.sync_copy(zs, e2_out.at[pl.ds(eb, _CHUNK)])
            return 0
        lax.fori_loop(0, nch, chunk, 0)

    return edge2


def kernel(x, edge_index, edge_attr, Wx, bx, We, be, W1, b1, a1, W2, b2, a2, W3, b3):
    N, IN = x.shape
    E = edge_attr.shape[0]
    HID = Wx.shape[1]
    H2 = HID // 2
    src = edge_index[0].astype(jnp.int32)
    dst = edge_index[1].astype(jnp.int32)
    bx2 = bx.reshape(1, HID)
    be2 = be.reshape(1, HID)
    b12 = b1.reshape(1, HID)
    b22 = b2.reshape(1, HID)
    b32 = b3.reshape(1, HID)
    a12 = a1.reshape(1, 1)
    a22 = a2.reshape(1, 1)

    RA = _row_block(N, (512, 400, 256, 200, 128, 80, 64, 40, 16, 8))
    RE = _row_block(E, (3200, 2560, 2048, 1600, 1280, 1024, 800, 640, 512, 256, 128))

    hf, h2 = pl.pallas_call(
        _lin_full_body,
        grid=(N // RA,),
        in_specs=[pl.BlockSpec((RA, IN), lambda i: (i, 0)),
                  pl.BlockSpec((IN, HID), lambda i: (0, 0)),
                  pl.BlockSpec((1, HID), lambda i: (0, 0))],
        out_specs=[pl.BlockSpec((RA, HID), lambda i: (i, 0)),
                   pl.BlockSpec((2, RA, H2), lambda i: (0, i, 0))],
        out_shape=[jax.ShapeDtypeStruct((N, HID), jnp.float32),
                   jax.ShapeDtypeStruct((2, N, H2), jnp.float32)],
    )(x, Wx, bx2)

    el2 = pl.pallas_call(
        _lin_half_body,
        grid=(E // RE,),
        in_specs=[pl.BlockSpec((RE, edge_attr.shape[1]), lambda i: (i, 0)),
                  pl.BlockSpec((edge_attr.shape[1], HID), lambda i: (0, 0)),
                  pl.BlockSpec((1, HID), lambda i: (0, 0))],
        out_specs=pl.BlockSpec((2, RE, H2), lambda i: (0, i, 0)),
        out_shape=jax.ShapeDtypeStruct((2, E, H2), jnp.float32),
    )(edge_attr, We, be2)

    es2, agg2 = _build_sc_edge1(N, E, H2)(
        h2.reshape(2 * N, H2), el2.reshape(2 * E, H2), src, dst)

    z = pl.pallas_call(
        _mlp_body,
        grid=(N // RA,),
        in_specs=[pl.BlockSpec((RA, HID), lambda i: (i, 0)),
                  pl.BlockSpec((2, RA, H2), lambda i: (0, i, 0)),
                  pl.BlockSpec((HID, HID), lambda i: (0, 0)),
                  pl.BlockSpec((1, HID), lambda i: (0, 0)),
                  pl.BlockSpec(memory_space=pltpu.SMEM),
                  pl.BlockSpec((HID, HID), lambda i: (0, 0)),
                  pl.BlockSpec((1, HID), lambda i: (0, 0)),
                  pl.BlockSpec(memory_space=pltpu.SMEM),
                  pl.BlockSpec((HID, HID), lambda i: (0, 0)),
                  pl.BlockSpec((1, HID), lambda i: (0, 0))],
        out_specs=pl.BlockSpec((RA, HID), lambda i: (i, 0)),
        out_shape=jax.ShapeDtypeStruct((N, HID), jnp.float32),
    )(hf, agg2.reshape(2, N, H2), W1, b12, a12, W2, b22, a22, W3, b32)

    e2 = _build_sc_edge2(N, E, HID)(z, es2, src, dst)
    return z, e2
